# baseline (speedup 1.0000x reference)
import functools

import jax
import jax.numpy as jnp
from jax import lax
from jax.experimental import pallas as pl
from jax.experimental.pallas import tpu as pltpu
from jax.experimental.pallas import tpu_sc as plsc

NC = 2
NS = 16
L = 16
FPT = 4
SW = 4000


def _sc_mesh():
    return plsc.VectorSubcoreMesh(core_axis_name="c", subcore_axis_name="s")


def _make_prep(N, E):
    EP = E // NS
    NCHK = EP // L
    HN = ((N + 255) // 256) * 256
    SPT = HN // NS
    assert E % (NS * L) == 0 and SPT % L == 0

    @functools.partial(
        pl.kernel,
        out_type=(
            jax.ShapeDtypeStruct((HN,), jnp.float32),
            jax.ShapeDtypeStruct((HN,), jnp.float32),
        ),
        mesh=_sc_mesh(),
        compiler_params=pltpu.CompilerParams(needs_layout_passes=False),
        scratch_types=[
            pltpu.VMEM((EP,), jnp.int32),
            pltpu.VMEM((HN,), jnp.float32),
            pltpu.VMEM((NS, SPT), jnp.float32),
            pltpu.VMEM((SPT,), jnp.float32),
            pltpu.VMEM_SHARED((NS, HN), jnp.float32),
        ],
    )
    def prep(eflat_hbm, z_hbm, ind_hbm, outd_hbm,
             ebuf, hv, rbuf, obuf, hsh):
        c = lax.axis_index("c")
        s = lax.axis_index("s")

        @pl.when(c == 0)
        def _():
            pltpu.sync_copy(eflat_hbm.at[pl.ds(E + s * EP, EP)], ebuf)

        @pl.when(c != 0)
        def _():
            pltpu.sync_copy(eflat_hbm.at[pl.ds(s * EP, EP)], ebuf)

        pltpu.sync_copy(z_hbm, hv)

        ones = jnp.ones((L,), jnp.float32)

        def body(i, _):
            v16 = ebuf[pl.ds(i * L, L)]
            plsc.addupdate_scatter(hv, [v16], ones)
            return 0

        lax.fori_loop(0, NCHK, body, 0)

        pltpu.sync_copy(hv, hsh.at[s])
        plsc.subcore_barrier()
        for t in range(NS):
            pltpu.sync_copy(hsh.at[t, pl.ds(SPT * s, SPT)], rbuf.at[t])

        def red(k, _):
            tot = jnp.zeros((L,), jnp.float32)
            for t in range(NS):
                tot = tot + rbuf[t, pl.ds(k * L, L)]
            obuf[pl.ds(k * L, L)] = tot
            return 0

        lax.fori_loop(0, SPT // L, red, 0)

        @pl.when(c == 0)
        def _():
            pltpu.sync_copy(obuf, ind_hbm.at[pl.ds(SPT * s, SPT)])

        @pl.when(c != 0)
        def _():
            pltpu.sync_copy(obuf, outd_hbm.at[pl.ds(SPT * s, SPT)])

    return prep


def _make_edge_pass(N, E, D):
    NP = ((N + 255) // 256) * 256
    NPASS = D // (NC * NS * FPT)
    NSTR = E // SW
    assert E % SW == 0 and SW % L == 0

    @functools.partial(
        pl.kernel,
        out_type=jax.ShapeDtypeStruct((D * NP,), jnp.float32),
        mesh=_sc_mesh(),
        compiler_params=pltpu.CompilerParams(needs_layout_passes=False),
        scratch_types=[
            pltpu.VMEM((SW,), jnp.int32),
            pltpu.VMEM((SW,), jnp.int32),
            pltpu.VMEM((SW,), jnp.int32),
            pltpu.VMEM((SW,), jnp.int32),
            pltpu.VMEM((FPT * NP,), jnp.float32),
            pltpu.VMEM((FPT * NP,), jnp.float32),
            pltpu.SemaphoreType.DMA,
            pltpu.SemaphoreType.DMA,
        ],
    )
    def edge_pass(gt_hbm, eflat_hbm, z_hbm, acc_hbm,
                  sbuf0, sbuf1, dbuf0, dbuf1, gslab, acc, sem0, sem1):
        c = lax.axis_index("c")
        s = lax.axis_index("s")
        w = c * NS + s
        sems = (sem0, sem1)
        sbufs = (sbuf0, sbuf1)
        dbufs = (dbuf0, dbuf1)

        for p in range(NPASS):
            fbase = p * (D // NPASS) + w * FPT
            pltpu.sync_copy(gt_hbm.at[pl.ds(fbase * NP, FPT * NP)], gslab)
            pltpu.sync_copy(z_hbm, acc)

            pltpu.async_copy(eflat_hbm.at[pl.ds(0, SW)], sbuf0, sem0)
            pltpu.async_copy(eflat_hbm.at[pl.ds(E, SW)], dbuf0, sem0)

            def spair(hg, _):
                for b in range(2):
                    gi = 2 * hg + b

                    @pl.when(gi < NSTR)
                    def _():
                        pltpu.make_async_copy(
                            eflat_hbm.at[pl.ds(0, SW)], sbufs[b],
                            sems[b]).wait()
                        pltpu.make_async_copy(
                            eflat_hbm.at[pl.ds(0, SW)], dbufs[b],
                            sems[b]).wait()

                        @pl.when(gi + 1 < NSTR)
                        def _():
                            nb = 1 - b
                            off = (gi + 1) * SW
                            pltpu.async_copy(eflat_hbm.at[pl.ds(off, SW)],
                                             sbufs[nb], sems[nb])
                            pltpu.async_copy(eflat_hbm.at[pl.ds(E + off, SW)],
                                             dbufs[nb], sems[nb])

                        @plsc.parallel_loop(0, SW // L, unroll=16)
                        def chunk(i):
                            s16 = sbufs[b][pl.ds(i * L, L)]
                            d16 = dbufs[b][pl.ds(i * L, L)]
                            for f in range(FPT):
                                v = plsc.load_gather(gslab, [s16 + f * NP])
                                plsc.addupdate_scatter(
                                    acc, [d16 + f * NP], v)
                return 0

            lax.fori_loop(0, (NSTR + 1) // 2, spair, 0)

            for f in range(FPT):
                pltpu.sync_copy(acc.at[pl.ds(f * NP, NP)],
                                acc_hbm.at[pl.ds((fbase + f) * NP, NP)])

    return edge_pass


def _tc_first(indeg, x, W, NP, D, BR):
    def body(ind_ref, x_ref, w_ref, o_ref):
        dis = lax.rsqrt(ind_ref[...] + 1.0)
        ht = lax.dot_general(w_ref[...], x_ref[...],
                             (((0,), (1,)), ((), ())),
                             preferred_element_type=jnp.float32)
        o_ref[...] = ht * dis

    return pl.pallas_call(
        body,
        grid=(NP // BR,),
        in_specs=[
            pl.BlockSpec((1, BR), lambda i: (0, i)),
            pl.BlockSpec((BR, D), lambda i: (i, 0)),
            pl.BlockSpec((D, D), lambda i: (0, 0)),
        ],
        out_specs=pl.BlockSpec((D, BR), lambda i: (0, i)),
        out_shape=jax.ShapeDtypeStruct((D, NP), jnp.float32),
    )(indeg, x, W)


def _tc_mid(indeg, accT, gT, b, W, NP, D, BR):
    def body(ind_ref, acc_ref, g_ref, b_ref, w_ref, o_ref):
        dis = lax.rsqrt(ind_ref[...] + 1.0)
        h = jnp.maximum(dis * (acc_ref[...] + g_ref[...]) + b_ref[...], 0.0)
        o_ref[...] = lax.dot_general(w_ref[...], h,
                                     (((0,), (0,)), ((), ())),
                                     preferred_element_type=jnp.float32) * dis

    return pl.pallas_call(
        body,
        grid=(NP // BR,),
        in_specs=[
            pl.BlockSpec((1, BR), lambda i: (0, i)),
            pl.BlockSpec((D, BR), lambda i: (0, i)),
            pl.BlockSpec((D, BR), lambda i: (0, i)),
            pl.BlockSpec((D, 1), lambda i: (0, 0)),
            pl.BlockSpec((D, D), lambda i: (0, 0)),
        ],
        out_specs=pl.BlockSpec((D, BR), lambda i: (0, i)),
        out_shape=jax.ShapeDtypeStruct((D, NP), jnp.float32),
    )(indeg, accT, gT, b, W)


def _tc_final(indeg, outdeg, accT, gT, b, NP, D, E, BR):
    inv_e = 1.0 / float(E)

    def body(ind_ref, od_ref, acc_ref, g_ref, b_ref, o_ref):
        i = pl.program_id(0)
        dis = lax.rsqrt(ind_ref[...] + 1.0)
        h = jnp.maximum(dis * (acc_ref[...] + g_ref[...]) + b_ref[...], 0.0)
        wv = od_ref[...] * inv_e
        part = lax.dot_general(wv, h, (((1,), (1,)), ((), ())),
                               preferred_element_type=jnp.float32)

        @pl.when(i == 0)
        def _():
            o_ref[...] = jnp.zeros_like(o_ref)

        o_ref[...] += part

    return pl.pallas_call(
        body,
        grid=(NP // BR,),
        in_specs=[
            pl.BlockSpec((1, BR), lambda i: (0, i)),
            pl.BlockSpec((1, BR), lambda i: (0, i)),
            pl.BlockSpec((D, BR), lambda i: (0, i)),
            pl.BlockSpec((D, BR), lambda i: (0, i)),
            pl.BlockSpec((D, 1), lambda i: (0, 0)),
        ],
        out_specs=pl.BlockSpec((1, D), lambda i: (0, 0)),
        out_shape=jax.ShapeDtypeStruct((1, D), jnp.float32),
    )(indeg, outdeg, accT, gT, b)


def kernel(x, edge_index, W1, b1, W2, b2):
    N, D = x.shape
    E = edge_index.shape[1]
    NP = ((N + 255) // 256) * 256
    BR = 1024
    assert NP % BR == 0

    prep = _make_prep(N, E)
    epass = _make_edge_pass(N, E, D)

    z_h = jnp.zeros((NP,), jnp.float32)
    z_a = jnp.zeros((FPT * NP,), jnp.float32)

    eflat = edge_index.reshape(2 * E)
    indeg_h, outdeg_h = prep(eflat, z_h)
    indeg = indeg_h.reshape(1, NP)
    outdeg = outdeg_h.reshape(1, NP)
    xp = jnp.pad(x, ((0, NP - N), (0, 0)))

    g1T = _tc_first(indeg, xp, W1, NP, D, BR)
    acc1T = epass(g1T.reshape(D * NP), eflat, z_a).reshape(D, NP)
    g2T = _tc_mid(indeg, acc1T, g1T, b1.reshape(D, 1), W2, NP, D, BR)
    acc2T = epass(g2T.reshape(D * NP), eflat, z_a).reshape(D, NP)
    out = _tc_final(indeg, outdeg, acc2T, g2T, b2.reshape(D, 1),
                    NP, D, E, BR)
    return out[0]


# speedup vs baseline: 1.1336x; 1.1336x over previous
import functools

import jax
import jax.numpy as jnp
from jax import lax
from jax.experimental import pallas as pl
from jax.experimental.pallas import tpu as pltpu
from jax.experimental.pallas import tpu_sc as plsc

NC = 2
NS = 16
L = 16
FPT = 4
SW = 4000


def _sc_mesh():
    return plsc.VectorSubcoreMesh(core_axis_name="c", subcore_axis_name="s")


def _make_prep(N, E):
    EP = E // NS
    NCHK = EP // L
    HN = ((N + 255) // 256) * 256
    SPT = HN // NS
    assert E % (NS * L) == 0 and SPT % L == 0

    @functools.partial(
        pl.kernel,
        out_type=(
            jax.ShapeDtypeStruct((HN,), jnp.float32),
            jax.ShapeDtypeStruct((HN,), jnp.float32),
        ),
        mesh=_sc_mesh(),
        compiler_params=pltpu.CompilerParams(needs_layout_passes=False),
        scratch_types=[
            pltpu.VMEM((EP,), jnp.int32),
            pltpu.VMEM((HN,), jnp.float32),
            pltpu.VMEM((NS, SPT), jnp.float32),
            pltpu.VMEM((SPT,), jnp.float32),
            pltpu.VMEM_SHARED((NS, HN), jnp.float32),
        ],
    )
    def prep(eflat_hbm, z_hbm, ind_hbm, outd_hbm,
             ebuf, hv, rbuf, obuf, hsh):
        c = lax.axis_index("c")
        s = lax.axis_index("s")

        @pl.when(c == 0)
        def _():
            pltpu.sync_copy(eflat_hbm.at[pl.ds(E + s * EP, EP)], ebuf)

        @pl.when(c != 0)
        def _():
            pltpu.sync_copy(eflat_hbm.at[pl.ds(s * EP, EP)], ebuf)

        pltpu.sync_copy(z_hbm, hv)

        ones = jnp.ones((L,), jnp.float32)

        def body(i, _):
            v16 = ebuf[pl.ds(i * L, L)]
            plsc.addupdate_scatter(hv, [v16], ones)
            return 0

        lax.fori_loop(0, NCHK, body, 0)

        pltpu.sync_copy(hv, hsh.at[s])
        plsc.subcore_barrier()
        for t in range(NS):
            pltpu.sync_copy(hsh.at[t, pl.ds(SPT * s, SPT)], rbuf.at[t])

        def red(k, _):
            tot = jnp.zeros((L,), jnp.float32)
            for t in range(NS):
                tot = tot + rbuf[t, pl.ds(k * L, L)]
            obuf[pl.ds(k * L, L)] = tot
            return 0

        lax.fori_loop(0, SPT // L, red, 0)

        @pl.when(c == 0)
        def _():
            pltpu.sync_copy(obuf, ind_hbm.at[pl.ds(SPT * s, SPT)])

        @pl.when(c != 0)
        def _():
            pltpu.sync_copy(obuf, outd_hbm.at[pl.ds(SPT * s, SPT)])

    return prep


def _make_edge_pass(N, E, D):
    NP = ((N + 255) // 256) * 256
    NPASS = D // (NC * NS * FPT)
    NSTR = E // SW
    assert E % SW == 0 and SW % L == 0

    @functools.partial(
        pl.kernel,
        out_type=jax.ShapeDtypeStruct((D * NP,), jnp.float32),
        mesh=_sc_mesh(),
        compiler_params=pltpu.CompilerParams(needs_layout_passes=False),
        scratch_types=[
            pltpu.VMEM((SW,), jnp.int32),
            pltpu.VMEM((SW,), jnp.int32),
            pltpu.VMEM((SW,), jnp.int32),
            pltpu.VMEM((SW,), jnp.int32),
            pltpu.VMEM((FPT * NP,), jnp.float32),
            pltpu.VMEM((FPT * NP,), jnp.float32),
            pltpu.SemaphoreType.DMA,
            pltpu.SemaphoreType.DMA,
        ],
    )
    def edge_pass(gt_hbm, eflat_hbm, z_hbm, acc_hbm,
                  sbuf0, sbuf1, dbuf0, dbuf1, gslab, acc, sem0, sem1):
        c = lax.axis_index("c")
        s = lax.axis_index("s")
        w = c * NS + s
        sems = (sem0, sem1)
        sbufs = (sbuf0, sbuf1)
        dbufs = (dbuf0, dbuf1)

        for p in range(NPASS):
            fbase = p * (D // NPASS) + w * FPT
            pltpu.sync_copy(gt_hbm.at[pl.ds(fbase * NP, FPT * NP)], gslab)
            pltpu.sync_copy(z_hbm, acc)

            pltpu.async_copy(eflat_hbm.at[pl.ds(0, SW)], sbuf0, sem0)
            pltpu.async_copy(eflat_hbm.at[pl.ds(E, SW)], dbuf0, sem0)

            def spair(hg, _):
                for b in range(2):
                    gi = 2 * hg + b

                    @pl.when(gi < NSTR)
                    def _():
                        pltpu.make_async_copy(
                            eflat_hbm.at[pl.ds(0, SW)], sbufs[b],
                            sems[b]).wait()
                        pltpu.make_async_copy(
                            eflat_hbm.at[pl.ds(0, SW)], dbufs[b],
                            sems[b]).wait()

                        @pl.when(gi + 1 < NSTR)
                        def _():
                            nb = 1 - b
                            off = (gi + 1) * SW
                            pltpu.async_copy(eflat_hbm.at[pl.ds(off, SW)],
                                             sbufs[nb], sems[nb])
                            pltpu.async_copy(eflat_hbm.at[pl.ds(E + off, SW)],
                                             dbufs[nb], sems[nb])

                        @plsc.parallel_loop(0, SW // L, unroll=8)
                        def chunk(i):
                            s16 = sbufs[b][pl.ds(i * L, L)]
                            d16 = dbufs[b][pl.ds(i * L, L)]
                            for f in range(FPT):
                                v = plsc.load_gather(gslab, [s16 + f * NP])
                                plsc.addupdate_scatter(
                                    acc, [d16 + f * NP], v)
                return 0

            lax.fori_loop(0, (NSTR + 1) // 2, spair, 0)

            for f in range(FPT):
                pltpu.sync_copy(acc.at[pl.ds(f * NP, NP)],
                                acc_hbm.at[pl.ds((fbase + f) * NP, NP)])

    return edge_pass


def _tc_first(indeg, x, W, NP, D, BR):
    def body(ind_ref, x_ref, w_ref, o_ref):
        dis = lax.rsqrt(ind_ref[...] + 1.0)
        ht = lax.dot_general(w_ref[...], x_ref[...],
                             (((0,), (1,)), ((), ())),
                             preferred_element_type=jnp.float32)
        o_ref[...] = ht * dis

    return pl.pallas_call(
        body,
        grid=(NP // BR,),
        in_specs=[
            pl.BlockSpec((1, BR), lambda i: (0, i)),
            pl.BlockSpec((BR, D), lambda i: (i, 0)),
            pl.BlockSpec((D, D), lambda i: (0, 0)),
        ],
        out_specs=pl.BlockSpec((D, BR), lambda i: (0, i)),
        out_shape=jax.ShapeDtypeStruct((D, NP), jnp.float32),
    )(indeg, x, W)


def _tc_mid(indeg, accT, gT, b, W, NP, D, BR):
    def body(ind_ref, acc_ref, g_ref, b_ref, w_ref, o_ref):
        dis = lax.rsqrt(ind_ref[...] + 1.0)
        h = jnp.maximum(dis * (acc_ref[...] + g_ref[...]) + b_ref[...], 0.0)
        o_ref[...] = lax.dot_general(w_ref[...], h,
                                     (((0,), (0,)), ((), ())),
                                     preferred_element_type=jnp.float32) * dis

    return pl.pallas_call(
        body,
        grid=(NP // BR,),
        in_specs=[
            pl.BlockSpec((1, BR), lambda i: (0, i)),
            pl.BlockSpec((D, BR), lambda i: (0, i)),
            pl.BlockSpec((D, BR), lambda i: (0, i)),
            pl.BlockSpec((D, 1), lambda i: (0, 0)),
            pl.BlockSpec((D, D), lambda i: (0, 0)),
        ],
        out_specs=pl.BlockSpec((D, BR), lambda i: (0, i)),
        out_shape=jax.ShapeDtypeStruct((D, NP), jnp.float32),
    )(indeg, accT, gT, b, W)


def _tc_final(indeg, outdeg, accT, gT, b, NP, D, E, BR):
    inv_e = 1.0 / float(E)

    def body(ind_ref, od_ref, acc_ref, g_ref, b_ref, o_ref):
        i = pl.program_id(0)
        dis = lax.rsqrt(ind_ref[...] + 1.0)
        h = jnp.maximum(dis * (acc_ref[...] + g_ref[...]) + b_ref[...], 0.0)
        wv = od_ref[...] * inv_e
        part = lax.dot_general(wv, h, (((1,), (1,)), ((), ())),
                               preferred_element_type=jnp.float32)

        @pl.when(i == 0)
        def _():
            o_ref[...] = jnp.zeros_like(o_ref)

        o_ref[...] += part

    return pl.pallas_call(
        body,
        grid=(NP // BR,),
        in_specs=[
            pl.BlockSpec((1, BR), lambda i: (0, i)),
            pl.BlockSpec((1, BR), lambda i: (0, i)),
            pl.BlockSpec((D, BR), lambda i: (0, i)),
            pl.BlockSpec((D, BR), lambda i: (0, i)),
            pl.BlockSpec((D, 1), lambda i: (0, 0)),
        ],
        out_specs=pl.BlockSpec((1, D), lambda i: (0, 0)),
        out_shape=jax.ShapeDtypeStruct((1, D), jnp.float32),
    )(indeg, outdeg, accT, gT, b)


def kernel(x, edge_index, W1, b1, W2, b2):
    N, D = x.shape
    E = edge_index.shape[1]
    NP = ((N + 255) // 256) * 256
    BR = 1024
    assert NP % BR == 0

    prep = _make_prep(N, E)
    epass = _make_edge_pass(N, E, D)

    z_h = jnp.zeros((NP,), jnp.float32)
    z_a = jnp.zeros((FPT * NP,), jnp.float32)

    eflat = edge_index.reshape(2 * E)
    indeg_h, outdeg_h = prep(eflat, z_h)
    indeg = indeg_h.reshape(1, NP)
    outdeg = outdeg_h.reshape(1, NP)
    xp = jnp.pad(x, ((0, NP - N), (0, 0)))

    g1T = _tc_first(indeg, xp, W1, NP, D, BR)
    acc1T = epass(g1T.reshape(D * NP), eflat, z_a).reshape(D, NP)
    g2T = _tc_mid(indeg, acc1T, g1T, b1.reshape(D, 1), W2, NP, D, BR)
    acc2T = epass(g2T.reshape(D * NP), eflat, z_a).reshape(D, NP)
    out = _tc_final(indeg, outdeg, acc2T, g2T, b2.reshape(D, 1),
                    NP, D, E, BR)
    return out[0]


# speedup vs baseline: 1.1522x; 1.0164x over previous
import functools

import jax
import jax.numpy as jnp
from jax import lax
from jax.experimental import pallas as pl
from jax.experimental.pallas import tpu as pltpu
from jax.experimental.pallas import tpu_sc as plsc

NC = 2
NS = 16
L = 16
FPT = 4
SW = 4000


def _sc_mesh():
    return plsc.VectorSubcoreMesh(core_axis_name="c", subcore_axis_name="s")


def _make_prep(N, E):
    EP = E // NS
    NCHK = EP // L
    HN = ((N + 255) // 256) * 256
    SPT = HN // NS
    assert E % (NS * L) == 0 and SPT % L == 0

    @functools.partial(
        pl.kernel,
        out_type=(
            jax.ShapeDtypeStruct((HN,), jnp.float32),
            jax.ShapeDtypeStruct((HN,), jnp.float32),
        ),
        mesh=_sc_mesh(),
        compiler_params=pltpu.CompilerParams(needs_layout_passes=False),
        scratch_types=[
            pltpu.VMEM((EP,), jnp.int32),
            pltpu.VMEM((HN,), jnp.float32),
            pltpu.VMEM((NS, SPT), jnp.float32),
            pltpu.VMEM((SPT,), jnp.float32),
            pltpu.VMEM_SHARED((NS, HN), jnp.float32),
        ],
    )
    def prep(eflat_hbm, z_hbm, ind_hbm, outd_hbm,
             ebuf, hv, rbuf, obuf, hsh):
        c = lax.axis_index("c")
        s = lax.axis_index("s")

        @pl.when(c == 0)
        def _():
            pltpu.sync_copy(eflat_hbm.at[pl.ds(E + s * EP, EP)], ebuf)

        @pl.when(c != 0)
        def _():
            pltpu.sync_copy(eflat_hbm.at[pl.ds(s * EP, EP)], ebuf)

        pltpu.sync_copy(z_hbm, hv)

        ones = jnp.ones((L,), jnp.float32)

        def body(i, _):
            v16 = ebuf[pl.ds(i * L, L)]
            plsc.addupdate_scatter(hv, [v16], ones)
            return 0

        lax.fori_loop(0, NCHK, body, 0)

        pltpu.sync_copy(hv, hsh.at[s])
        plsc.subcore_barrier()
        for t in range(NS):
            pltpu.sync_copy(hsh.at[t, pl.ds(SPT * s, SPT)], rbuf.at[t])

        def red(k, _):
            tot = jnp.zeros((L,), jnp.float32)
            for t in range(NS):
                tot = tot + rbuf[t, pl.ds(k * L, L)]
            obuf[pl.ds(k * L, L)] = tot
            return 0

        lax.fori_loop(0, SPT // L, red, 0)

        @pl.when(c == 0)
        def _():
            pltpu.sync_copy(obuf, ind_hbm.at[pl.ds(SPT * s, SPT)])

        @pl.when(c != 0)
        def _():
            pltpu.sync_copy(obuf, outd_hbm.at[pl.ds(SPT * s, SPT)])

    return prep


def _make_edge_pass(N, E, D):
    NP = ((N + 255) // 256) * 256
    NPASS = D // (NC * NS * FPT)
    NSTR = E // SW
    assert E % SW == 0 and SW % L == 0

    @functools.partial(
        pl.kernel,
        out_type=jax.ShapeDtypeStruct((D * NP,), jnp.float32),
        mesh=_sc_mesh(),
        compiler_params=pltpu.CompilerParams(needs_layout_passes=False),
        scratch_types=[
            pltpu.VMEM((SW,), jnp.int32),
            pltpu.VMEM((SW,), jnp.int32),
            pltpu.VMEM((SW,), jnp.int32),
            pltpu.VMEM((SW,), jnp.int32),
            pltpu.VMEM((FPT * NP,), jnp.float32),
            pltpu.VMEM((FPT * NP,), jnp.float32),
            pltpu.SemaphoreType.DMA,
            pltpu.SemaphoreType.DMA,
        ],
    )
    def edge_pass(gt_hbm, eflat_hbm, z_hbm, acc_hbm,
                  sbuf0, sbuf1, dbuf0, dbuf1, gslab, acc, sem0, sem1):
        c = lax.axis_index("c")
        s = lax.axis_index("s")
        w = c * NS + s
        sems = (sem0, sem1)
        sbufs = (sbuf0, sbuf1)
        dbufs = (dbuf0, dbuf1)

        for p in range(NPASS):
            fbase = p * (D // NPASS) + w * FPT
            pltpu.sync_copy(gt_hbm.at[pl.ds(fbase * NP, FPT * NP)], gslab)
            pltpu.sync_copy(z_hbm, acc)

            pltpu.async_copy(eflat_hbm.at[pl.ds(0, SW)], sbuf0, sem0)
            pltpu.async_copy(eflat_hbm.at[pl.ds(E, SW)], dbuf0, sem0)

            def spair(hg, _):
                for b in range(2):
                    gi = 2 * hg + b

                    @pl.when(gi < NSTR)
                    def _():
                        pltpu.make_async_copy(
                            eflat_hbm.at[pl.ds(0, SW)], sbufs[b],
                            sems[b]).wait()
                        pltpu.make_async_copy(
                            eflat_hbm.at[pl.ds(0, SW)], dbufs[b],
                            sems[b]).wait()

                        @pl.when(gi + 1 < NSTR)
                        def _():
                            nb = 1 - b
                            off = (gi + 1) * SW
                            pltpu.async_copy(eflat_hbm.at[pl.ds(off, SW)],
                                             sbufs[nb], sems[nb])
                            pltpu.async_copy(eflat_hbm.at[pl.ds(E + off, SW)],
                                             dbufs[nb], sems[nb])

                        @plsc.parallel_loop(0, SW // L, unroll=10)
                        def chunk(i):
                            s16 = sbufs[b][pl.ds(i * L, L)]
                            d16 = dbufs[b][pl.ds(i * L, L)]
                            for f in range(FPT):
                                v = plsc.load_gather(gslab, [s16 + f * NP])
                                plsc.addupdate_scatter(
                                    acc, [d16 + f * NP], v)
                return 0

            lax.fori_loop(0, (NSTR + 1) // 2, spair, 0)

            for f in range(FPT):
                pltpu.sync_copy(acc.at[pl.ds(f * NP, NP)],
                                acc_hbm.at[pl.ds((fbase + f) * NP, NP)])

    return edge_pass


def _tc_first(indeg, x, W, NP, D, BR):
    def body(ind_ref, x_ref, w_ref, o_ref):
        dis = lax.rsqrt(ind_ref[...] + 1.0)
        ht = lax.dot_general(w_ref[...], x_ref[...],
                             (((0,), (1,)), ((), ())),
                             preferred_element_type=jnp.float32)
        o_ref[...] = ht * dis

    return pl.pallas_call(
        body,
        grid=(NP // BR,),
        in_specs=[
            pl.BlockSpec((1, BR), lambda i: (0, i)),
            pl.BlockSpec((BR, D), lambda i: (i, 0)),
            pl.BlockSpec((D, D), lambda i: (0, 0)),
        ],
        out_specs=pl.BlockSpec((D, BR), lambda i: (0, i)),
        out_shape=jax.ShapeDtypeStruct((D, NP), jnp.float32),
    )(indeg, x, W)


def _tc_mid(indeg, accT, gT, b, W, NP, D, BR):
    def body(ind_ref, acc_ref, g_ref, b_ref, w_ref, o_ref):
        dis = lax.rsqrt(ind_ref[...] + 1.0)
        h = jnp.maximum(dis * (acc_ref[...] + g_ref[...]) + b_ref[...], 0.0)
        o_ref[...] = lax.dot_general(w_ref[...], h,
                                     (((0,), (0,)), ((), ())),
                                     preferred_element_type=jnp.float32) * dis

    return pl.pallas_call(
        body,
        grid=(NP // BR,),
        in_specs=[
            pl.BlockSpec((1, BR), lambda i: (0, i)),
            pl.BlockSpec((D, BR), lambda i: (0, i)),
            pl.BlockSpec((D, BR), lambda i: (0, i)),
            pl.BlockSpec((D, 1), lambda i: (0, 0)),
            pl.BlockSpec((D, D), lambda i: (0, 0)),
        ],
        out_specs=pl.BlockSpec((D, BR), lambda i: (0, i)),
        out_shape=jax.ShapeDtypeStruct((D, NP), jnp.float32),
    )(indeg, accT, gT, b, W)


def _tc_final(indeg, outdeg, accT, gT, b, NP, D, E, BR):
    inv_e = 1.0 / float(E)

    def body(ind_ref, od_ref, acc_ref, g_ref, b_ref, o_ref):
        i = pl.program_id(0)
        dis = lax.rsqrt(ind_ref[...] + 1.0)
        h = jnp.maximum(dis * (acc_ref[...] + g_ref[...]) + b_ref[...], 0.0)
        wv = od_ref[...] * inv_e
        part = lax.dot_general(wv, h, (((1,), (1,)), ((), ())),
                               preferred_element_type=jnp.float32)

        @pl.when(i == 0)
        def _():
            o_ref[...] = jnp.zeros_like(o_ref)

        o_ref[...] += part

    return pl.pallas_call(
        body,
        grid=(NP // BR,),
        in_specs=[
            pl.BlockSpec((1, BR), lambda i: (0, i)),
            pl.BlockSpec((1, BR), lambda i: (0, i)),
            pl.BlockSpec((D, BR), lambda i: (0, i)),
            pl.BlockSpec((D, BR), lambda i: (0, i)),
            pl.BlockSpec((D, 1), lambda i: (0, 0)),
        ],
        out_specs=pl.BlockSpec((1, D), lambda i: (0, 0)),
        out_shape=jax.ShapeDtypeStruct((1, D), jnp.float32),
    )(indeg, outdeg, accT, gT, b)


def kernel(x, edge_index, W1, b1, W2, b2):
    N, D = x.shape
    E = edge_index.shape[1]
    NP = ((N + 255) // 256) * 256
    BR = 1024
    assert NP % BR == 0

    prep = _make_prep(N, E)
    epass = _make_edge_pass(N, E, D)

    z_h = jnp.zeros((NP,), jnp.float32)
    z_a = jnp.zeros((FPT * NP,), jnp.float32)

    eflat = edge_index.reshape(2 * E)
    indeg_h, outdeg_h = prep(eflat, z_h)
    indeg = indeg_h.reshape(1, NP)
    outdeg = outdeg_h.reshape(1, NP)
    xp = jnp.pad(x, ((0, NP - N), (0, 0)))

    g1T = _tc_first(indeg, xp, W1, NP, D, BR)
    acc1T = epass(g1T.reshape(D * NP), eflat, z_a).reshape(D, NP)
    g2T = _tc_mid(indeg, acc1T, g1T, b1.reshape(D, 1), W2, NP, D, BR)
    acc2T = epass(g2T.reshape(D * NP), eflat, z_a).reshape(D, NP)
    out = _tc_final(indeg, outdeg, acc2T, g2T, b2.reshape(D, 1),
                    NP, D, E, BR)
    return out[0]
